# baseline (speedup 1.0000x reference)
import functools

import jax
import jax.numpy as jnp
from jax import lax
from jax.experimental import pallas as pl
from jax.experimental.pallas import tpu as pltpu
from jax.experimental.pallas import tpu_sc as plsc

N_ATOMS = 50000
M_NBR = 12
F = 64
N_CRYSTALS = 512
N_EDGES = N_ATOMS * M_NBR

NW = 32
GATHER_PAD = 622592
IDX_ROWS = GATHER_PAD // 128
ROWS_PER_W = IDX_ROWS // NW
NSLOT = 4
NITER = ROWS_PER_W // NSLOT

TA = 400
TE = TA * M_NBR
GRID_E = N_ATOMS // TA
TA_V = 2000
GRID_V = N_ATOMS // TA_V
TP = 400
GRID_P = N_ATOMS // TP
EPS = 1e-5


def _softplus(x):
    return jnp.logaddexp(x, 0.0)


def _sc_gather(table, idx2d):
    mesh = plsc.VectorSubcoreMesh(core_axis_name="c", subcore_axis_name="s")

    @functools.partial(
        pl.kernel,
        mesh=mesh,
        out_type=jax.ShapeDtypeStruct((GATHER_PAD, 2 * F), jnp.float32),
        scratch_types=(
            [pltpu.VMEM((ROWS_PER_W, 128), jnp.int32)]
            + [pltpu.VMEM((128, 2 * F), jnp.float32)] * NSLOT
            + [pltpu.SemaphoreType.DMA] * (2 * NSLOT)
        ),
    )
    def k(table_hbm, idx_hbm, out_hbm, idx_v, *scratch):
        row_bufs = scratch[:NSLOT]
        sg = scratch[NSLOT:2 * NSLOT]
        sw = scratch[2 * NSLOT:]
        wid = lax.axis_index("s") * 2 + lax.axis_index("c")
        row0 = wid * ROWS_PER_W

        def out_region(c):
            return out_hbm.at[pl.ds((row0 + c) * 128, 128), :]

        pltpu.sync_copy(idx_hbm.at[pl.ds(row0, ROWS_PER_W), :], idx_v)

        for b in range(NSLOT):
            pltpu.async_copy(row_bufs[b], out_region(b), sw[b])

        def body(t, carry):
            handles = []
            for b in range(NSLOT):
                c = t * NSLOT + b
                pltpu.make_async_copy(row_bufs[b], out_region(c), sw[b]).wait()
                handles.append(pltpu.async_copy(
                    table_hbm.at[idx_v.at[c]], row_bufs[b], sg[b]))
            for b in range(NSLOT):
                c = t * NSLOT + b
                handles[b].wait()
                pltpu.async_copy(row_bufs[b], out_region(c), sw[b])
            return carry

        lax.fori_loop(0, NITER, body, 0)
        for b in range(NSLOT):
            pltpu.make_async_copy(row_bufs[b], out_region(b), sw[b]).wait()

    return k(table, idx2d)


def _embed_body(a_ref, w_ref, b_ref, o_ref):
    o_ref[...] = jnp.dot(a_ref[...], w_ref[...],
                         preferred_element_type=jnp.float32) + b_ref[...]


def _embed(atom_fea, W_embed, b_embed):
    n, k = atom_fea.shape
    ta = 2000
    return pl.pallas_call(
        _embed_body,
        grid=(n // ta,),
        in_specs=[
            pl.BlockSpec((ta, k), lambda i: (i, 0)),
            pl.BlockSpec((k, F), lambda i: (0, 0)),
            pl.BlockSpec((1, F), lambda i: (0, 0)),
        ],
        out_specs=pl.BlockSpec((ta, F), lambda i: (i, 0)),
        out_shape=jax.ShapeDtypeStruct((n, F), jnp.float32),
    )(atom_fea, W_embed, b_embed.reshape(1, F))


def _vn_body(v_ref, wn_ref, o_ref):
    o_ref[...] = jnp.dot(v_ref[...], wn_ref[...],
                         preferred_element_type=jnp.float32)


def _vn(v, wn):
    return pl.pallas_call(
        _vn_body,
        grid=(GRID_V,),
        in_specs=[
            pl.BlockSpec((TA_V, F), lambda i: (i, 0)),
            pl.BlockSpec((F, 2 * F), lambda i: (0, 0)),
        ],
        out_specs=pl.BlockSpec((TA_V, 2 * F), lambda i: (i, 0)),
        out_shape=jax.ShapeDtypeStruct((N_ATOMS, 2 * F), jnp.float32),
    )(v, wn)


def _edge_y(v_ref, g_ref, nf_ref, ws_ref, we_ref, bf_ref):
    vs = jnp.dot(v_ref[...], ws_ref[...], preferred_element_type=jnp.float32)
    vs = jnp.broadcast_to(vs[:, None, :], (TA, M_NBR, 2 * F)).reshape(TE, 2 * F)
    y = vs + g_ref[...]
    nf = nf_ref[...].reshape(TE, 41)
    y = y + jnp.dot(nf, we_ref[...], preferred_element_type=jnp.float32)
    return y + bf_ref[...]


def _passA_body(v_ref, g_ref, nf_ref, ws_ref, we_ref, bf_ref, acc_ref):
    @pl.when(pl.program_id(0) == 0)
    def _():
        acc_ref[...] = jnp.zeros_like(acc_ref)

    y = _edge_y(v_ref, g_ref, nf_ref, ws_ref, we_ref, bf_ref)
    acc_ref[0:1, :] += jnp.sum(y, axis=0, keepdims=True)
    acc_ref[1:2, :] += jnp.sum(y * y, axis=0, keepdims=True)


def _edge_in_specs():
    return [
        pl.BlockSpec((TA, F), lambda i: (i, 0)),
        pl.BlockSpec((TE, 2 * F), lambda i: (i, 0)),
        pl.BlockSpec((TA, M_NBR, 41), lambda i: (i, 0, 0)),
        pl.BlockSpec((F, 2 * F), lambda i: (0, 0)),
        pl.BlockSpec((41, 2 * F), lambda i: (0, 0)),
        pl.BlockSpec((1, 2 * F), lambda i: (0, 0)),
    ]


def _passA(v, g, nf_flat, ws, we, bf):
    return pl.pallas_call(
        _passA_body,
        grid=(GRID_E,),
        in_specs=_edge_in_specs(),
        out_specs=pl.BlockSpec((8, 2 * F), lambda i: (0, 0)),
        out_shape=jax.ShapeDtypeStruct((8, 2 * F), jnp.float32),
    )(v, g, nf_flat, ws, we, bf)


def _passB_body(v_ref, g_ref, nf_ref, ws_ref, we_ref, bf_ref,
                acc_ref, g1_ref, bb1_ref, ns_ref, acc2_ref):
    @pl.when(pl.program_id(0) == 0)
    def _():
        acc2_ref[...] = jnp.zeros_like(acc2_ref)

    inv_n = 1.0 / N_EDGES
    mu = acc_ref[0:1, :] * inv_n
    var = acc_ref[1:2, :] * inv_n - mu * mu
    scale = g1_ref[...] * lax.rsqrt(var + EPS)
    shift = bb1_ref[...] - mu * scale

    y = _edge_y(v_ref, g_ref, nf_ref, ws_ref, we_ref, bf_ref)
    y = y * scale + shift
    filt = jax.nn.sigmoid(y[:, :F])
    core = _softplus(y[:, F:])
    prod = (filt * core).reshape(TA, M_NBR, F)
    s = jnp.sum(prod, axis=1)
    ns_ref[...] = s
    row = jnp.concatenate(
        [jnp.sum(s, axis=0, keepdims=True),
         jnp.sum(s * s, axis=0, keepdims=True)], axis=1)
    acc2_ref[0:1, :] += row


def _passB(v, g, nf_flat, ws, we, bf, acc, g1, bb1):
    return pl.pallas_call(
        _passB_body,
        grid=(GRID_E,),
        in_specs=_edge_in_specs() + [
            pl.BlockSpec((8, 2 * F), lambda i: (0, 0)),
            pl.BlockSpec((1, 2 * F), lambda i: (0, 0)),
            pl.BlockSpec((1, 2 * F), lambda i: (0, 0)),
        ],
        out_specs=[
            pl.BlockSpec((TA, F), lambda i: (i, 0)),
            pl.BlockSpec((8, 2 * F), lambda i: (0, 0)),
        ],
        out_shape=[
            jax.ShapeDtypeStruct((N_ATOMS, F), jnp.float32),
            jax.ShapeDtypeStruct((8, 2 * F), jnp.float32),
        ],
    )(v, g, nf_flat, ws, we, bf, acc, g1, bb1)


def _passC_body(v_ref, ns_ref, acc2_ref, g2_ref, bb2_ref, o_ref):
    inv_n = 1.0 / N_ATOMS
    mu = acc2_ref[0:1, :F] * inv_n
    var = acc2_ref[0:1, F:] * inv_n - mu * mu
    scale = g2_ref[...] * lax.rsqrt(var + EPS)
    shift = bb2_ref[...] - mu * scale
    o_ref[...] = _softplus(v_ref[...] + ns_ref[...] * scale + shift)


def _passC(v, ns, acc2, g2, bb2):
    return pl.pallas_call(
        _passC_body,
        grid=(GRID_V,),
        in_specs=[
            pl.BlockSpec((TA_V, F), lambda i: (i, 0)),
            pl.BlockSpec((TA_V, F), lambda i: (i, 0)),
            pl.BlockSpec((8, 2 * F), lambda i: (0, 0)),
            pl.BlockSpec((1, F), lambda i: (0, 0)),
            pl.BlockSpec((1, F), lambda i: (0, 0)),
        ],
        out_specs=pl.BlockSpec((TA_V, F), lambda i: (i, 0)),
        out_shape=jax.ShapeDtypeStruct((N_ATOMS, F), jnp.float32),
    )(v, ns, acc2, g2, bb2)


def _pool_body(ids_ref, v_ref, acc_ref):
    @pl.when(pl.program_id(0) == 0)
    def _():
        acc_ref[...] = jnp.zeros_like(acc_ref)

    ids = ids_ref[0, 0, :]
    iota = lax.broadcasted_iota(jnp.int32, (N_CRYSTALS, TP), 0)
    onehot = (iota == ids[None, :]).astype(jnp.float32)
    v_ext = jnp.concatenate(
        [v_ref[...], jnp.ones((TP, F), jnp.float32)], axis=1)
    acc_ref[...] += jnp.dot(onehot, v_ext, preferred_element_type=jnp.float32)


def _pool(ids3d, v):
    return pl.pallas_call(
        _pool_body,
        grid=(GRID_P,),
        in_specs=[
            pl.BlockSpec((1, 1, TP), lambda i: (i, 0, 0)),
            pl.BlockSpec((TP, F), lambda i: (i, 0)),
        ],
        out_specs=pl.BlockSpec((N_CRYSTALS, 2 * F), lambda i: (0, 0)),
        out_shape=jax.ShapeDtypeStruct((N_CRYSTALS, 2 * F), jnp.float32),
    )(ids3d, v)


def _head_body(acc_ref, wp1_ref, bp1_ref, wp2_ref, bp2_ref, o_ref):
    sums = acc_ref[:, :F]
    counts = acc_ref[:, F:]
    crys = sums / jnp.maximum(counts, 1.0)
    h = jnp.maximum(
        jnp.dot(crys, wp1_ref[...], preferred_element_type=jnp.float32)
        + bp1_ref[...], 0.0)
    o_ref[...] = jnp.dot(h, wp2_ref[...],
                         preferred_element_type=jnp.float32) + bp2_ref[...]


def _head(acc, Wp1, bp1, Wp2, bp2):
    return pl.pallas_call(
        _head_body,
        in_specs=[pl.BlockSpec(acc.shape, lambda: (0, 0)),
                  pl.BlockSpec((F, F), lambda: (0, 0)),
                  pl.BlockSpec((1, F), lambda: (0, 0)),
                  pl.BlockSpec((F, F), lambda: (0, 0)),
                  pl.BlockSpec((1, F), lambda: (0, 0))],
        out_specs=pl.BlockSpec((N_CRYSTALS, F), lambda: (0, 0)),
        out_shape=jax.ShapeDtypeStruct((N_CRYSTALS, F), jnp.float32),
    )(acc, Wp1, bp1.reshape(1, F), Wp2, bp2.reshape(1, F))


def kernel(atom_fea, nbr_fea, nbr_fea_idx, crystal_atom_idx, W_embed, b_embed,
           Wf0, bf0, g1_0, bb1_0, g2_0, bb2_0,
           Wf1, bf1, g1_1, bb1_1, g2_1, bb2_1,
           Wf2, bf2, g1_2, bb1_2, g2_2, bb2_2,
           Wp1, bp1, Wp2, bp2):
    layers = [(Wf0, bf0, g1_0, bb1_0, g2_0, bb2_0),
              (Wf1, bf1, g1_1, bb1_1, g2_1, bb2_1),
              (Wf2, bf2, g1_2, bb1_2, g2_2, bb2_2)]

    idx_flat = nbr_fea_idx.reshape(-1)
    idx_pad = jnp.pad(idx_flat, (0, GATHER_PAD - N_EDGES)).reshape(IDX_ROWS, 128)
    nf_flat = nbr_fea
    ids3d = crystal_atom_idx.reshape(GRID_P, 1, TP)

    v = _embed(atom_fea, W_embed, b_embed)

    for (Wf, bf, g1, bb1, g2, bb2) in layers:
        ws = Wf[:F]
        wn = Wf[F:2 * F]
        we = Wf[2 * F:]
        bf2d = bf.reshape(1, 2 * F)
        vn = _vn(v, wn)
        g = _sc_gather(vn, idx_pad)
        acc = _passA(v, g, nf_flat, ws, we, bf2d)
        ns, acc2 = _passB(v, g, nf_flat, ws, we, bf2d, acc,
                          g1.reshape(1, 2 * F), bb1.reshape(1, 2 * F))
        v = _passC(v, ns, acc2, g2.reshape(1, F), bb2.reshape(1, F))

    acc_pool = _pool(ids3d, v)
    return _head(acc_pool, Wp1, bp1, Wp2, bp2)


# speedup vs baseline: 1.1662x; 1.1662x over previous
import functools

import jax
import jax.numpy as jnp
from jax import lax
from jax.experimental import pallas as pl
from jax.experimental.pallas import tpu as pltpu
from jax.experimental.pallas import tpu_sc as plsc

N_ATOMS = 50000
M_NBR = 12
F = 64
N_CRYSTALS = 512
N_EDGES = N_ATOMS * M_NBR

NW = 32
GATHER_PAD = 614400
IDX_ROWS = GATHER_PAD // 128
ROWS_PER_W = IDX_ROWS // NW
CHUNK_ROWS = 2
NSLOT = 3
NITER = ROWS_PER_W // (CHUNK_ROWS * NSLOT)

TA = 400
TE = TA * M_NBR
GRID_E = N_ATOMS // TA
TA_V = 2000
GRID_V = N_ATOMS // TA_V
TP = 400
GRID_P = N_ATOMS // TP
EPS = 1e-5


def _softplus(x):
    return jnp.logaddexp(x, 0.0)


def _sc_gather(table, idx2d):
    mesh = plsc.VectorSubcoreMesh(core_axis_name="c", subcore_axis_name="s")

    nrow = CHUNK_ROWS * 128

    @functools.partial(
        pl.kernel,
        mesh=mesh,
        out_type=jax.ShapeDtypeStruct((GATHER_PAD, 2 * F), jnp.float32),
        scratch_types=(
            [pltpu.VMEM((CHUNK_ROWS, 128), jnp.int32)] * NSLOT
            + [pltpu.VMEM((nrow, 2 * F), jnp.float32)] * NSLOT
            + [pltpu.SemaphoreType.DMA] * (2 * NSLOT)
        ),
    )
    def k(table_hbm, idx_hbm, out_hbm, *scratch):
        idx_bufs = scratch[:NSLOT]
        row_bufs = scratch[NSLOT:2 * NSLOT]
        sg = scratch[2 * NSLOT:3 * NSLOT]
        sw = scratch[3 * NSLOT:]
        wid = lax.axis_index("s") * 2 + lax.axis_index("c")
        row0 = wid * ROWS_PER_W

        def out_region(c):
            return out_hbm.at[pl.ds((row0 + c * CHUNK_ROWS) * 128, nrow), :]

        for b in range(NSLOT):
            pltpu.async_copy(row_bufs[b], out_region(b), sw[b])

        def body(t, carry):
            handles = []
            for b in range(NSLOT):
                c = t * NSLOT + b
                pltpu.make_async_copy(row_bufs[b], out_region(c), sw[b]).wait()
                pltpu.sync_copy(
                    idx_hbm.at[pl.ds(row0 + c * CHUNK_ROWS, CHUNK_ROWS), :],
                    idx_bufs[b])
                handles.append([
                    pltpu.async_copy(
                        table_hbm.at[idx_bufs[b].at[j]],
                        row_bufs[b].at[pl.ds(j * 128, 128), :],
                        sg[b])
                    for j in range(CHUNK_ROWS)])
            for b in range(NSLOT):
                c = t * NSLOT + b
                for h in handles[b]:
                    h.wait()
                pltpu.async_copy(row_bufs[b], out_region(c), sw[b])
            return carry

        lax.fori_loop(0, NITER, body, 0)
        for b in range(NSLOT):
            pltpu.make_async_copy(row_bufs[b], out_region(b), sw[b]).wait()

    return k(table, idx2d)


def _embed_body(a_ref, w_ref, b_ref, o_ref):
    o_ref[...] = jnp.dot(a_ref[...], w_ref[...],
                         preferred_element_type=jnp.float32) + b_ref[...]


def _embed(atom_fea, W_embed, b_embed):
    n, k = atom_fea.shape
    ta = 2000
    return pl.pallas_call(
        _embed_body,
        grid=(n // ta,),
        in_specs=[
            pl.BlockSpec((ta, k), lambda i: (i, 0)),
            pl.BlockSpec((k, F), lambda i: (0, 0)),
            pl.BlockSpec((1, F), lambda i: (0, 0)),
        ],
        out_specs=pl.BlockSpec((ta, F), lambda i: (i, 0)),
        out_shape=jax.ShapeDtypeStruct((n, F), jnp.float32),
    )(atom_fea, W_embed, b_embed.reshape(1, F))


def _vn_body(v_ref, wn_ref, o_ref):
    o_ref[...] = jnp.dot(v_ref[...], wn_ref[...],
                         preferred_element_type=jnp.float32)


def _vn(v, wn):
    return pl.pallas_call(
        _vn_body,
        grid=(GRID_V,),
        in_specs=[
            pl.BlockSpec((TA_V, F), lambda i: (i, 0)),
            pl.BlockSpec((F, 2 * F), lambda i: (0, 0)),
        ],
        out_specs=pl.BlockSpec((TA_V, 2 * F), lambda i: (i, 0)),
        out_shape=jax.ShapeDtypeStruct((N_ATOMS, 2 * F), jnp.float32),
    )(v, wn)


def _edge_y(v_ref, g_ref, nf_ref, ws_ref, we_ref, bf_ref):
    vs = jnp.dot(v_ref[...], ws_ref[...], preferred_element_type=jnp.float32)
    vs = jnp.broadcast_to(vs[:, None, :], (TA, M_NBR, 2 * F)).reshape(TE, 2 * F)
    y = vs + g_ref[...]
    nf = nf_ref[...].reshape(TE, 41)
    y = y + jnp.dot(nf, we_ref[...], preferred_element_type=jnp.float32)
    return y + bf_ref[...]


def _passA_body(v_ref, g_ref, nf_ref, ws_ref, we_ref, bf_ref, acc_ref):
    @pl.when(pl.program_id(0) == 0)
    def _():
        acc_ref[...] = jnp.zeros_like(acc_ref)

    y = _edge_y(v_ref, g_ref, nf_ref, ws_ref, we_ref, bf_ref)
    acc_ref[0:1, :] += jnp.sum(y, axis=0, keepdims=True)
    acc_ref[1:2, :] += jnp.sum(y * y, axis=0, keepdims=True)


def _edge_in_specs():
    return [
        pl.BlockSpec((TA, F), lambda i: (i, 0)),
        pl.BlockSpec((TE, 2 * F), lambda i: (i, 0)),
        pl.BlockSpec((TA, M_NBR, 41), lambda i: (i, 0, 0)),
        pl.BlockSpec((F, 2 * F), lambda i: (0, 0)),
        pl.BlockSpec((41, 2 * F), lambda i: (0, 0)),
        pl.BlockSpec((1, 2 * F), lambda i: (0, 0)),
    ]


def _passA(v, g, nf_flat, ws, we, bf):
    return pl.pallas_call(
        _passA_body,
        grid=(GRID_E,),
        in_specs=_edge_in_specs(),
        out_specs=pl.BlockSpec((8, 2 * F), lambda i: (0, 0)),
        out_shape=jax.ShapeDtypeStruct((8, 2 * F), jnp.float32),
    )(v, g, nf_flat, ws, we, bf)


def _passB_body(v_ref, g_ref, nf_ref, ws_ref, we_ref, bf_ref,
                acc_ref, g1_ref, bb1_ref, ns_ref, acc2_ref):
    @pl.when(pl.program_id(0) == 0)
    def _():
        acc2_ref[...] = jnp.zeros_like(acc2_ref)

    inv_n = 1.0 / N_EDGES
    mu = acc_ref[0:1, :] * inv_n
    var = acc_ref[1:2, :] * inv_n - mu * mu
    scale = g1_ref[...] * lax.rsqrt(var + EPS)
    shift = bb1_ref[...] - mu * scale

    y = _edge_y(v_ref, g_ref, nf_ref, ws_ref, we_ref, bf_ref)
    y = y * scale + shift
    filt = jax.nn.sigmoid(y[:, :F])
    core = _softplus(y[:, F:])
    prod = (filt * core).reshape(TA, M_NBR, F)
    s = jnp.sum(prod, axis=1)
    ns_ref[...] = s
    row = jnp.concatenate(
        [jnp.sum(s, axis=0, keepdims=True),
         jnp.sum(s * s, axis=0, keepdims=True)], axis=1)
    acc2_ref[0:1, :] += row


def _passB(v, g, nf_flat, ws, we, bf, acc, g1, bb1):
    return pl.pallas_call(
        _passB_body,
        grid=(GRID_E,),
        in_specs=_edge_in_specs() + [
            pl.BlockSpec((8, 2 * F), lambda i: (0, 0)),
            pl.BlockSpec((1, 2 * F), lambda i: (0, 0)),
            pl.BlockSpec((1, 2 * F), lambda i: (0, 0)),
        ],
        out_specs=[
            pl.BlockSpec((TA, F), lambda i: (i, 0)),
            pl.BlockSpec((8, 2 * F), lambda i: (0, 0)),
        ],
        out_shape=[
            jax.ShapeDtypeStruct((N_ATOMS, F), jnp.float32),
            jax.ShapeDtypeStruct((8, 2 * F), jnp.float32),
        ],
    )(v, g, nf_flat, ws, we, bf, acc, g1, bb1)


def _passC_body(v_ref, ns_ref, acc2_ref, g2_ref, bb2_ref, o_ref):
    inv_n = 1.0 / N_ATOMS
    mu = acc2_ref[0:1, :F] * inv_n
    var = acc2_ref[0:1, F:] * inv_n - mu * mu
    scale = g2_ref[...] * lax.rsqrt(var + EPS)
    shift = bb2_ref[...] - mu * scale
    o_ref[...] = _softplus(v_ref[...] + ns_ref[...] * scale + shift)


def _passC(v, ns, acc2, g2, bb2):
    return pl.pallas_call(
        _passC_body,
        grid=(GRID_V,),
        in_specs=[
            pl.BlockSpec((TA_V, F), lambda i: (i, 0)),
            pl.BlockSpec((TA_V, F), lambda i: (i, 0)),
            pl.BlockSpec((8, 2 * F), lambda i: (0, 0)),
            pl.BlockSpec((1, F), lambda i: (0, 0)),
            pl.BlockSpec((1, F), lambda i: (0, 0)),
        ],
        out_specs=pl.BlockSpec((TA_V, F), lambda i: (i, 0)),
        out_shape=jax.ShapeDtypeStruct((N_ATOMS, F), jnp.float32),
    )(v, ns, acc2, g2, bb2)


def _pool_body(ids_ref, v_ref, acc_ref):
    @pl.when(pl.program_id(0) == 0)
    def _():
        acc_ref[...] = jnp.zeros_like(acc_ref)

    ids = ids_ref[0, 0, :]
    iota = lax.broadcasted_iota(jnp.int32, (N_CRYSTALS, TP), 0)
    onehot = (iota == ids[None, :]).astype(jnp.float32)
    v_ext = jnp.concatenate(
        [v_ref[...], jnp.ones((TP, F), jnp.float32)], axis=1)
    acc_ref[...] += jnp.dot(onehot, v_ext, preferred_element_type=jnp.float32)


def _pool(ids3d, v):
    return pl.pallas_call(
        _pool_body,
        grid=(GRID_P,),
        in_specs=[
            pl.BlockSpec((1, 1, TP), lambda i: (i, 0, 0)),
            pl.BlockSpec((TP, F), lambda i: (i, 0)),
        ],
        out_specs=pl.BlockSpec((N_CRYSTALS, 2 * F), lambda i: (0, 0)),
        out_shape=jax.ShapeDtypeStruct((N_CRYSTALS, 2 * F), jnp.float32),
    )(ids3d, v)


def _head_body(acc_ref, wp1_ref, bp1_ref, wp2_ref, bp2_ref, o_ref):
    sums = acc_ref[:, :F]
    counts = acc_ref[:, F:]
    crys = sums / jnp.maximum(counts, 1.0)
    h = jnp.maximum(
        jnp.dot(crys, wp1_ref[...], preferred_element_type=jnp.float32)
        + bp1_ref[...], 0.0)
    o_ref[...] = jnp.dot(h, wp2_ref[...],
                         preferred_element_type=jnp.float32) + bp2_ref[...]


def _head(acc, Wp1, bp1, Wp2, bp2):
    return pl.pallas_call(
        _head_body,
        in_specs=[pl.BlockSpec(acc.shape, lambda: (0, 0)),
                  pl.BlockSpec((F, F), lambda: (0, 0)),
                  pl.BlockSpec((1, F), lambda: (0, 0)),
                  pl.BlockSpec((F, F), lambda: (0, 0)),
                  pl.BlockSpec((1, F), lambda: (0, 0))],
        out_specs=pl.BlockSpec((N_CRYSTALS, F), lambda: (0, 0)),
        out_shape=jax.ShapeDtypeStruct((N_CRYSTALS, F), jnp.float32),
    )(acc, Wp1, bp1.reshape(1, F), Wp2, bp2.reshape(1, F))


def kernel(atom_fea, nbr_fea, nbr_fea_idx, crystal_atom_idx, W_embed, b_embed,
           Wf0, bf0, g1_0, bb1_0, g2_0, bb2_0,
           Wf1, bf1, g1_1, bb1_1, g2_1, bb2_1,
           Wf2, bf2, g1_2, bb1_2, g2_2, bb2_2,
           Wp1, bp1, Wp2, bp2):
    layers = [(Wf0, bf0, g1_0, bb1_0, g2_0, bb2_0),
              (Wf1, bf1, g1_1, bb1_1, g2_1, bb2_1),
              (Wf2, bf2, g1_2, bb1_2, g2_2, bb2_2)]

    idx_flat = nbr_fea_idx.reshape(-1)
    idx_pad = jnp.pad(idx_flat, (0, GATHER_PAD - N_EDGES)).reshape(IDX_ROWS, 128)
    nf_flat = nbr_fea
    ids3d = crystal_atom_idx.reshape(GRID_P, 1, TP)

    v = _embed(atom_fea, W_embed, b_embed)

    for (Wf, bf, g1, bb1, g2, bb2) in layers:
        ws = Wf[:F]
        wn = Wf[F:2 * F]
        we = Wf[2 * F:]
        bf2d = bf.reshape(1, 2 * F)
        vn = _vn(v, wn)
        g = _sc_gather(vn, idx_pad)
        acc = _passA(v, g, nf_flat, ws, we, bf2d)
        ns, acc2 = _passB(v, g, nf_flat, ws, we, bf2d, acc,
                          g1.reshape(1, 2 * F), bb1.reshape(1, 2 * F))
        v = _passC(v, ns, acc2, g2.reshape(1, F), bb2.reshape(1, F))

    acc_pool = _pool(ids3d, v)
    return _head(acc_pool, Wp1, bp1, Wp2, bp2)
